# baseline (speedup 1.0000x reference)
import functools

import jax
import jax.numpy as jnp
from jax import lax
from jax.experimental import pallas as pl
from jax.experimental.pallas import tpu as pltpu
from jax.experimental.pallas import tpu_sc as plsc

NUM_CORES = 2
NUM_SUBCORES = 16
NUM_WORKERS = NUM_CORES * NUM_SUBCORES

CHUNK = 160
NBUF = 4


def _make_gather(total, dim, dtype):
    assert total % (NUM_WORKERS * CHUNK * NBUF) == 0
    per_w = total // NUM_WORKERS
    n_chunks = per_w // CHUNK
    n_outer = n_chunks // NBUF
    mesh = plsc.VectorSubcoreMesh(core_axis_name="c", subcore_axis_name="s")

    @functools.partial(
        pl.kernel,
        mesh=mesh,
        out_type=jax.ShapeDtypeStruct((total, dim), dtype),
        scratch_types=[
            pltpu.VMEM((per_w,), jnp.int32),
            [pltpu.VMEM((CHUNK, dim), dtype) for _ in range(NBUF)],
            [pltpu.SemaphoreType.DMA for _ in range(NBUF)],
            [pltpu.SemaphoreType.DMA for _ in range(NBUF)],
        ],
    )
    def gather_kernel(idx_hbm, table_hbm, out_hbm, idx_v, rows, gsems, wsems):
        wid = lax.axis_index("s") * NUM_CORES + lax.axis_index("c")
        base = pl.multiple_of(wid * per_w, CHUNK)
        pltpu.sync_copy(idx_hbm.at[pl.ds(base, per_w)], idx_v)

        def start_gather(g, b):
            off = pl.multiple_of(g * CHUNK, CHUNK)
            pltpu.async_copy(
                table_hbm.at[idx_v.at[pl.ds(off, CHUNK)]], rows[b], gsems[b]
            )

        def wait_gather(b):
            pltpu.make_async_copy(
                table_hbm.at[idx_v.at[pl.ds(0, CHUNK)]], rows[b], gsems[b]
            ).wait()

        def wait_write(b):
            pltpu.make_async_copy(
                rows[b], out_hbm.at[pl.ds(0, CHUNK)], wsems[b]
            ).wait()

        start_gather(0, 0)
        start_gather(1, 1)

        def step(g, b):
            wait_gather(b)

            b2 = (b + 2) % NBUF

            @pl.when(g + 2 < n_chunks)
            def _():
                @pl.when(g >= 2)
                def _():
                    wait_write(b2)

                start_gather(g + 2, b2)

            off = pl.multiple_of(g * CHUNK, CHUNK)
            pltpu.async_copy(
                rows[b], out_hbm.at[pl.ds(base + off, CHUNK)], wsems[b]
            )

        def outer(i, carry):
            for b in range(NBUF):
                step(i * NBUF + b, b)
            return carry

        lax.fori_loop(0, n_outer, outer, 0)

        for b in range(NBUF):
            wait_write(b)

    return gather_kernel


def kernel(token_ids, embedding_matrix):
    batch, seq = token_ids.shape
    num_rows, dim = embedding_matrix.shape
    pad_dim = 128
    flat_ids = token_ids.reshape(batch * seq)
    table128 = jnp.pad(embedding_matrix, ((0, 0), (0, pad_dim - dim)))
    fn = _make_gather(batch * seq, pad_dim, embedding_matrix.dtype)
    out = fn(flat_ids, table128)
    return out[:, :dim].reshape(batch, seq, dim)
